# initial kernel scaffold (unmeasured)
import jax
import jax.numpy as jnp
from jax import lax
from jax.experimental import pallas as pl
from jax.experimental.pallas import tpu as pltpu


def kernel(Q, K, V):
    b, sq, h, d = Q.shape
    skv = K.shape[1]
    scale = d ** -0.5

    def partial_body(q_ref, k_ref, v_ref, o_ref, m_ref, l_ref):
        for hi in range(h):
            q = q_ref[0, :, hi, :]
            k = k_ref[0, :, hi, :]
            v = v_ref[0, :, hi, :]
            s = lax.dot_general(
                q, k, (((1,), (1,)), ((), ())),
                preferred_element_type=jnp.float32,
            ) * scale
            m = jnp.max(s, axis=1, keepdims=True)
            p = jnp.exp(s - m)
            l = jnp.sum(p, axis=1, keepdims=True)
            o = lax.dot_general(
                p, v, (((1,), (0,)), ((), ())),
                preferred_element_type=jnp.float32,
            )
            o_ref[0, :, hi, :] = o
            m_ref[0, :, hi:hi + 1] = m
            l_ref[0, :, hi:hi + 1] = l

    o_p, m_p, l_p = pl.pallas_call(
        partial_body,
        grid=(b,),
        in_specs=[
            pl.BlockSpec((1, sq, h, d), lambda bi: (bi, 0, 0, 0)),
            pl.BlockSpec((1, skv, h, d), lambda bi: (bi, 0, 0, 0)),
            pl.BlockSpec((1, skv, h, d), lambda bi: (bi, 0, 0, 0)),
        ],
        out_specs=[
            pl.BlockSpec((1, sq, h, d), lambda bi: (bi, 0, 0, 0)),
            pl.BlockSpec((1, sq, h), lambda bi: (bi, 0, 0)),
            pl.BlockSpec((1, sq, h), lambda bi: (bi, 0, 0)),
        ],
        out_shape=[
            jax.ShapeDtypeStruct((b, sq, h, d), jnp.float32),
            jax.ShapeDtypeStruct((b, sq, h), jnp.float32),
            jax.ShapeDtypeStruct((b, sq, h), jnp.float32),
        ],
    )(Q, K, V)

    def combine_body(o_ref, m_ref, l_ref, out_ref,
                     o_rx, m_rx, l_rx, send_sems, recv_sems):
        my_x = lax.axis_index("x")
        my_y = lax.axis_index("y")
        my_z = lax.axis_index("z")
        partner = (my_x, 1 - my_y, my_z)

        barrier = pltpu.get_barrier_semaphore()
        pl.semaphore_signal(barrier, inc=1, device_id=partner,
                            device_id_type=pl.DeviceIdType.MESH)
        pl.semaphore_wait(barrier, 1)

        copies = []
        for i, (src, dst) in enumerate(
            ((o_ref, o_rx), (m_ref, m_rx), (l_ref, l_rx))
        ):
            c = pltpu.make_async_remote_copy(
                src_ref=src, dst_ref=dst,
                send_sem=send_sems.at[i], recv_sem=recv_sems.at[i],
                device_id=partner, device_id_type=pl.DeviceIdType.MESH,
            )
            c.start()
            copies.append(c)
        for c in copies:
            c.wait()

        m_a = m_ref[...]
        m_b = m_rx[...]
        m_n = jnp.maximum(m_a, m_b)
        ca = jnp.exp(m_a - m_n)
        cb = jnp.exp(m_b - m_n)
        l_n = ca * l_ref[...] + cb * l_rx[...]
        wa = (ca / l_n)[..., None]
        wb = (cb / l_n)[..., None]
        out_ref[...] = o_ref[...] * wa + o_rx[...] * wb

    return pl.pallas_call(
        combine_body,
        in_specs=[pl.BlockSpec(memory_space=pltpu.VMEM)] * 3,
        out_specs=pl.BlockSpec(memory_space=pltpu.VMEM),
        out_shape=jax.ShapeDtypeStruct((b, sq, h, d), jnp.float32),
        scratch_shapes=[
            pltpu.VMEM((b, sq, h, d), jnp.float32),
            pltpu.VMEM((b, sq, h), jnp.float32),
            pltpu.VMEM((b, sq, h), jnp.float32),
            pltpu.SemaphoreType.DMA((3,)),
            pltpu.SemaphoreType.DMA((3,)),
        ],
        compiler_params=pltpu.CompilerParams(collective_id=0),
    )(o_p, m_p, l_p)


# baseline (device time: 97025 ns/iter reference)
import jax
import jax.numpy as jnp
from jax import lax
from jax.experimental import pallas as pl
from jax.experimental.pallas import tpu as pltpu


def kernel(Q, K, V):
    b, sq, h, d = Q.shape
    skv = K.shape[1]
    scale = d ** -0.5

    def partial_body(q_ref, k_ref, v_ref, o_ref, m_ref, l_ref):
        for hi in range(h):
            q = q_ref[0, :, hi, :]
            k = k_ref[0, :, hi, :]
            v = v_ref[0, :, hi, :]
            s = lax.dot_general(
                q, k, (((1,), (1,)), ((), ())),
                preferred_element_type=jnp.float32,
            ) * scale
            m = jnp.max(s, axis=1, keepdims=True)
            p = jnp.exp(s - m)
            l = jnp.sum(p, axis=1, keepdims=True)
            o = lax.dot_general(
                p, v, (((1,), (0,)), ((), ())),
                preferred_element_type=jnp.float32,
            )
            o_ref[0, :, hi, :] = o
            m_ref[0, :, hi:hi + 1] = m
            l_ref[0, :, hi:hi + 1] = l

    o_p, m_p, l_p = pl.pallas_call(
        partial_body,
        grid=(b,),
        in_specs=[
            pl.BlockSpec((1, sq, h, d), lambda bi: (bi, 0, 0, 0)),
            pl.BlockSpec((1, skv, h, d), lambda bi: (bi, 0, 0, 0)),
            pl.BlockSpec((1, skv, h, d), lambda bi: (bi, 0, 0, 0)),
        ],
        out_specs=[
            pl.BlockSpec((1, sq, h, d), lambda bi: (bi, 0, 0, 0)),
            pl.BlockSpec((1, sq, h), lambda bi: (bi, 0, 0)),
            pl.BlockSpec((1, sq, h), lambda bi: (bi, 0, 0)),
        ],
        out_shape=[
            jax.ShapeDtypeStruct((b, sq, h, d), jnp.float32),
            jax.ShapeDtypeStruct((b, sq, h), jnp.float32),
            jax.ShapeDtypeStruct((b, sq, h), jnp.float32),
        ],
        compiler_params=pltpu.CompilerParams(
            vmem_limit_bytes=48 * 1024 * 1024,
        ),
    )(Q, K, V)

    def combine_body(o_ref, m_ref, l_ref, out_ref,
                     o_rx, m_rx, l_rx, send_sems, recv_sems):
        my_x = lax.axis_index("x")
        my_y = lax.axis_index("y")
        my_z = lax.axis_index("z")
        partner = (my_x, 1 - my_y, my_z)

        barrier = pltpu.get_barrier_semaphore()
        pl.semaphore_signal(barrier, inc=1, device_id=partner,
                            device_id_type=pl.DeviceIdType.MESH)
        pl.semaphore_wait(barrier, 1)

        copies = []
        for i, (src, dst) in enumerate(
            ((o_ref, o_rx), (m_ref, m_rx), (l_ref, l_rx))
        ):
            c = pltpu.make_async_remote_copy(
                src_ref=src, dst_ref=dst,
                send_sem=send_sems.at[i], recv_sem=recv_sems.at[i],
                device_id=partner, device_id_type=pl.DeviceIdType.MESH,
            )
            c.start()
            copies.append(c)
        for c in copies:
            c.wait()

        m_a = m_ref[...]
        m_b = m_rx[...]
        m_n = jnp.maximum(m_a, m_b)
        ca = jnp.exp(m_a - m_n)
        cb = jnp.exp(m_b - m_n)
        l_n = ca * l_ref[...] + cb * l_rx[...]
        wa = (ca / l_n)[..., None]
        wb = (cb / l_n)[..., None]
        out_ref[...] = o_ref[...] * wa + o_rx[...] * wb

    return pl.pallas_call(
        combine_body,
        in_specs=[pl.BlockSpec(memory_space=pltpu.VMEM)] * 3,
        out_specs=pl.BlockSpec(memory_space=pltpu.VMEM),
        out_shape=jax.ShapeDtypeStruct((b, sq, h, d), jnp.float32),
        scratch_shapes=[
            pltpu.VMEM((b, sq, h, d), jnp.float32),
            pltpu.VMEM((b, sq, h), jnp.float32),
            pltpu.VMEM((b, sq, h), jnp.float32),
            pltpu.SemaphoreType.DMA((3,)),
            pltpu.SemaphoreType.DMA((3,)),
        ],
        compiler_params=pltpu.CompilerParams(collective_id=0),
    )(o_p, m_p, l_p)
